# baseline (speedup 1.0000x reference)
import functools

import jax
import jax.numpy as jnp
from jax import lax
from jax.experimental import pallas as pl
from jax.experimental.pallas import tpu as pltpu
from jax.experimental.pallas import tpu_sc as plsc

B = 16384
D = 64
NC = 2
NS = 16
NW = NC * NS
BPW = B // NW
GCHUNK = 128
NCHUNK = BPW // GCHUNK
LANES = 16
Y_LO = 0.0
Y_HI = 5.5


def _make_sc_kernel():
    mesh = plsc.VectorSubcoreMesh(core_axis_name="c", subcore_axis_name="s")

    @functools.partial(
        pl.kernel,
        mesh=mesh,
        compiler_params=pltpu.CompilerParams(
            needs_layout_passes=False, use_tc_tiling_on_sc=False),
        out_type=jax.ShapeDtypeStruct((B,), jnp.float32),
        scratch_types=[
            pltpu.VMEM((NCHUNK, GCHUNK), jnp.int32),
            pltpu.VMEM((NCHUNK, GCHUNK), jnp.int32),
            pltpu.VMEM((BPW, D), jnp.float32),
            pltpu.VMEM((BPW, D), jnp.float32),
            pltpu.VMEM((BPW, 1), jnp.float32),
            pltpu.VMEM((BPW, 1), jnp.float32),
            pltpu.VMEM((BPW,), jnp.float32),
            pltpu.SemaphoreType.DMA,
        ],
    )
    def sc_kernel(users_hbm, items_hbm, uw_hbm, iw_hbm, ub_hbm, ib_hbm,
                  out_hbm, idx_u, idx_i, rows_u, rows_i, bias_u, bias_i,
                  out_v, sem):
        wid = lax.axis_index("s") * NC + lax.axis_index("c")
        base = wid * BPW

        pltpu.sync_copy(users_hbm.at[wid], idx_u)
        pltpu.sync_copy(items_hbm.at[wid], idx_i)

        copies = []
        for j in range(NCHUNK):
            sl = pl.ds(j * GCHUNK, GCHUNK)
            copies.append(pltpu.make_async_copy(uw_hbm.at[idx_u.at[j]], rows_u.at[sl], sem))
            copies.append(pltpu.make_async_copy(iw_hbm.at[idx_i.at[j]], rows_i.at[sl], sem))
            copies.append(pltpu.make_async_copy(ub_hbm.at[idx_u.at[j]], bias_u.at[sl], sem))
            copies.append(pltpu.make_async_copy(ib_hbm.at[idx_i.at[j]], bias_i.at[sl], sem))
        for c in copies:
            c.start()
        for c in copies:
            c.wait()

        lanes = lax.iota(jnp.int32, LANES)
        zeros = jnp.zeros((LANES,), jnp.int32)

        def group_body(g, _):
            row_idx = lanes + g * LANES
            acc = jnp.zeros((LANES,), jnp.float32)
            for f in range(D):
                col = jnp.full((LANES,), f, jnp.int32)
                uv = plsc.load_gather(rows_u, [row_idx, col])
                iv = plsc.load_gather(rows_i, [row_idx, col])
                acc = acc + uv * iv
            bu = plsc.load_gather(bias_u, [row_idx, zeros])
            bi = plsc.load_gather(bias_i, [row_idx, zeros])
            res = acc + bu + bi
            y = (Y_HI - Y_LO) / (1.0 + jnp.exp(-res)) + Y_LO
            plsc.store_scatter(out_v, [row_idx], y)
            return 0

        lax.fori_loop(0, BPW // LANES, group_body, 0)

        pltpu.sync_copy(out_v, out_hbm.at[pl.ds(base, BPW)])

    return sc_kernel


_SC_KERNEL = _make_sc_kernel()


@jax.jit
def kernel(x, u_weight, i_weight, u_bias, i_bias):
    users = x[:, 0].astype(jnp.int32).reshape(NW, NCHUNK, GCHUNK)
    items = x[:, 1].astype(jnp.int32).reshape(NW, NCHUNK, GCHUNK)
    return _SC_KERNEL(users, items, u_weight, i_weight, u_bias, i_bias)


# speedup vs baseline: 5.3130x; 5.3130x over previous
import functools

import jax
import jax.numpy as jnp
from jax import lax
from jax.experimental import pallas as pl
from jax.experimental.pallas import tpu as pltpu
from jax.experimental.pallas import tpu_sc as plsc

B = 16384
D = 64
W = 1000000
NC = 2
NS = 16
NW = NC * NS
BPW = B // NW
G = 16
NG = BPW // G
RF = 2
NR = G // RF
LANES = 16
WIN = 128
Y_LO = 0.0
Y_HI = 5.5


def _make_sc_kernel():
    mesh = plsc.VectorSubcoreMesh(core_axis_name="c", subcore_axis_name="s")

    @functools.partial(
        pl.kernel,
        mesh=mesh,
        compiler_params=pltpu.CompilerParams(
            needs_layout_passes=False, use_tc_tiling_on_sc=True),
        out_type=jax.ShapeDtypeStruct((B,), jnp.float32),
        scratch_types=[
            pltpu.VMEM((BPW,), jnp.int32),
            pltpu.VMEM((BPW,), jnp.int32),
            pltpu.VMEM((2, D, RF * WIN), jnp.float32),
            pltpu.VMEM((2, D, RF * WIN), jnp.float32),
            pltpu.VMEM((G * WIN,), jnp.float32),
            pltpu.VMEM((G * WIN,), jnp.float32),
            pltpu.VMEM((BPW,), jnp.float32),
            pltpu.SemaphoreType.DMA,
            pltpu.SemaphoreType.DMA,
        ],
    )
    def sc_kernel(users_hbm, items_hbm, uwt_hbm, iwt_hbm, ub_hbm, ib_hbm,
                  out_hbm, idx_u, idx_i, au, ai, bu, bi, out_v, sem_w, sem_b):
        wid = lax.axis_index("s") * NC + lax.axis_index("c")
        base = wid * BPW

        pltpu.sync_copy(users_hbm.at[wid], idx_u)
        pltpu.sync_copy(items_hbm.at[wid], idx_i)

        lanes = lax.iota(jnp.int32, LANES)

        def fire_round(p, lu16, li16):
            wcopies = []
            bcopies = []
            for k in range(RF):
                el = RF * p + k
                lu = pl.multiple_of(lu16[el], WIN)
                li = pl.multiple_of(li16[el], WIN)
                slab = p % 2
                dst = pl.ds(k * WIN, WIN)
                wcopies.append(pltpu.make_async_copy(
                    uwt_hbm.at[:, pl.ds(lu, WIN)],
                    au.at[slab].at[:, dst], sem_w))
                wcopies.append(pltpu.make_async_copy(
                    iwt_hbm.at[:, pl.ds(li, WIN)],
                    ai.at[slab].at[:, dst], sem_w))
                bcopies.append(pltpu.make_async_copy(
                    ub_hbm.at[pl.ds(lu, WIN)],
                    bu.at[pl.ds(el * WIN, WIN)], sem_b))
                bcopies.append(pltpu.make_async_copy(
                    ib_hbm.at[pl.ds(li, WIN)],
                    bi.at[pl.ds(el * WIN, WIN)], sem_b))
            for c in wcopies + bcopies:
                c.start()
            return wcopies, bcopies

        def extract_round(p, acc, colu, coli):
            slab = p % 2
            mp = (lanes // RF) == p
            part = jnp.zeros((LANES,), jnp.float32)
            for f in range(D):
                rowf = jnp.full((LANES,), f, jnp.int32)
                uv = plsc.load_gather(au.at[slab], [rowf, colu])
                iv = plsc.load_gather(ai.at[slab], [rowf, coli])
                part = part + uv * iv
            return acc + jnp.where(mp, part, 0.0)

        def group_body(g, _):
            e0 = g * G
            vu16 = idx_u[pl.ds(e0, LANES)]
            vi16 = idx_i[pl.ds(e0, LANES)]
            lu16 = vu16 & -WIN
            li16 = vi16 & -WIN
            colu = (vu16 & (WIN - 1)) + (lanes % RF) * WIN
            coli = (vi16 & (WIN - 1)) + (lanes % RF) * WIN

            acc = jnp.zeros((LANES,), jnp.float32)
            all_bias = []
            prev, pbias = fire_round(0, lu16, li16)
            all_bias += pbias
            for p in range(1, NR):
                cur, cbias = fire_round(p, lu16, li16)
                all_bias += cbias
                for c in prev:
                    c.wait()
                acc = extract_round(p - 1, acc, colu, coli)
                prev = cur
            for c in prev:
                c.wait()
            acc = extract_round(NR - 1, acc, colu, coli)

            for c in all_bias:
                c.wait()

            colb_u = (vu16 & (WIN - 1)) + lanes * WIN
            colb_i = (vi16 & (WIN - 1)) + lanes * WIN
            buv = plsc.load_gather(bu, [colb_u])
            biv = plsc.load_gather(bi, [colb_i])
            res = acc + buv + biv
            y = (Y_HI - Y_LO) / (1.0 + jnp.exp(-res)) + Y_LO
            out_v[pl.ds(e0, LANES)] = y
            return 0

        lax.fori_loop(0, NG, group_body, 0)

        pltpu.sync_copy(out_v, out_hbm.at[pl.ds(base, BPW)])

    return sc_kernel


_SC_KERNEL = _make_sc_kernel()


@jax.jit
def kernel(x, u_weight, i_weight, u_bias, i_bias):
    users = x[:, 0].astype(jnp.int32).reshape(NW, BPW)
    items = x[:, 1].astype(jnp.int32).reshape(NW, BPW)
    return _SC_KERNEL(users, items, u_weight.T, i_weight.T,
                      u_bias.reshape(-1), i_bias.reshape(-1))
